# baseline (speedup 1.0000x reference)
import functools

import jax
import jax.numpy as jnp
from jax import lax
from jax.experimental import pallas as pl
from jax.experimental.pallas import tpu as pltpu
from jax.experimental.pallas import tpu_sc as plsc

_NUM_ENVS = 512
_MAX_LENGTH = 2048
_FEAT = 128
_SAMPLE_B = 1024
_MBL = 8
_L = 16
_NW = 32
_BPW = _SAMPLE_B // _NW
_RPW = _BPW * _MBL
_NCHUNK = 8
_CROWS = _RPW // _NCHUNK
_CSMP = _BPW // _NCHUNK


def _replay_body(mem_hbm, exp_hbm, wp_hbm, env_hbm, pos_hbm,
                 out_hbm, wout_hbm,
                 env_v, pos_v, wp_v, base_v, idx_v, rows_v, exp_v, o_v,
                 ones_v, sems):
    cid = lax.axis_index("c")
    sid = lax.axis_index("s")
    wid = sid * 2 + cid
    base = wid * _BPW

    cp_env = pltpu.async_copy(env_hbm.at[pl.ds(base, _BPW)], env_v, sems.at[0])
    cp_pos = pltpu.async_copy(pos_hbm.at[pl.ds(base, _BPW)], pos_v, sems.at[1])
    cp_wp = pltpu.async_copy(wp_hbm, wp_v, sems.at[2])

    lane = lax.iota(jnp.int32, _L)
    one16 = jnp.full((_L,), 1.0, jnp.float32)
    ones_v[pl.ds(0, _L)] = one16
    ones_v[pl.ds(_L, _L)] = one16
    cp_w = pltpu.async_copy(ones_v, wout_hbm.at[pl.ds(base, _BPW)], sems.at[4])

    cp_env.wait()
    cp_exp = pltpu.async_copy(exp_hbm.at[env_v], exp_v,
                              sems.at[5 + 2 * _NCHUNK])
    cp_pos.wait()

    for h in range(_BPW // _L):
        e16 = env_v[pl.ds(h * _L, _L)]
        p16 = pos_v[pl.ds(h * _L, _L)]
        base_v[pl.ds(h * _L, _L)] = e16 * _MAX_LENGTH + p16

    def idx_step(i, carry):
        j = lane + i * _L
        b = j >> 3
        t = j & 7
        idx_v[i // (_CROWS // _L), pl.ds((i % (_CROWS // _L)) * _L, _L)] = (
            plsc.load_gather(base_v, [b]) + t)
        return carry

    lax.fori_loop(0, _RPW // _L, idx_step, 0)
    gcps = []
    for k in range(_NCHUNK):
        gcps.append(pltpu.async_copy(
            mem_hbm.at[idx_v.at[k]],
            rows_v.at[pl.ds(k * _CROWS, _CROWS)], sems.at[5 + k]))

    cp_wp.wait()
    nhit = []
    for h in range(_BPW // _L):
        e16 = env_v[pl.ds(h * _L, _L)]
        p16 = pos_v[pl.ds(h * _L, _L)]
        o16 = plsc.load_gather(wp_v, [e16]) - p16
        o_v[pl.ds(_L + h * _L, _L)] = o16
        hit16 = (o16 >= 0) & (o16 < _MBL)
        nhit.append(jnp.max(jnp.where(hit16, 1, 0)))
    cp_exp.wait()

    def fixup(k):
        def body():
            def one_sample(smp, carry):
                sel = jnp.full((_L,), smp + _L, jnp.int32)
                o_spl = plsc.load_gather(o_v, [sel])
                hit = (o_spl >= 0) & (o_spl < _MBL)
                row_idx = smp * _MBL + jnp.where(hit, o_spl, 0)
                row_sel = jnp.full((_L,), smp, jnp.int32)
                for cc in range(_FEAT // _L):
                    col = lane + cc * _L
                    vals = plsc.load_gather(exp_v, [row_sel, col])
                    plsc.store_scatter(rows_v, [row_idx, col], vals, mask=hit)
                return carry
            lax.fori_loop(k * _CSMP, (k + 1) * _CSMP, one_sample, 0)
        pl.when(nhit[(k * _CSMP) // _L] > 0)(body)

    out_cps = []
    for k in range(_NCHUNK):
        gcps[k].wait()
        fixup(k)
        out_cps.append(pltpu.async_copy(
            rows_v.at[pl.ds(k * _CROWS, _CROWS)],
            out_hbm.at[pl.ds(wid * _RPW + k * _CROWS, _CROWS)],
            sems.at[5 + _NCHUNK + k]))
    for cp in out_cps:
        cp.wait()
    cp_w.wait()


@jax.jit
def _replay(mem2d, exp, write_pos, env_ids, positions):
    mesh = plsc.VectorSubcoreMesh(core_axis_name="c", subcore_axis_name="s",
                                  num_cores=2, num_subcores=16)
    run = pl.kernel(
        _replay_body,
        out_type=(
            jax.ShapeDtypeStruct((_SAMPLE_B * _MBL, _FEAT), jnp.float32),
            jax.ShapeDtypeStruct((_SAMPLE_B,), jnp.float32),
        ),
        mesh=mesh,
        compiler_params=pltpu.CompilerParams(needs_layout_passes=False,
                                             skip_device_barrier=True),
        scratch_types=[
            pltpu.VMEM((_BPW,), jnp.int32),
            pltpu.VMEM((_BPW,), jnp.int32),
            pltpu.VMEM((_NUM_ENVS,), jnp.int32),
            pltpu.VMEM((_BPW,), jnp.int32),
            pltpu.VMEM((_NCHUNK, _CROWS), jnp.int32),
            pltpu.VMEM((_RPW, _FEAT), jnp.float32),
            pltpu.VMEM((_BPW, _FEAT), jnp.float32),
            pltpu.VMEM((_L + _BPW,), jnp.int32),
            pltpu.VMEM((_BPW,), jnp.float32),
            pltpu.SemaphoreType.DMA((6 + 2 * _NCHUNK,)),
        ],
    )
    return run(mem2d, exp, write_pos, env_ids, positions)


def kernel(mem, exp, write_pos, env_ids, positions, mini_batch_length):
    mem2d = mem.reshape(_NUM_ENVS * _MAX_LENGTH, _FEAT)
    env = env_ids.astype(jnp.int32)
    wp = write_pos.astype(jnp.int32)
    pos = (positions + (mini_batch_length - _MBL)).astype(jnp.int32)
    samples, importance_weights = _replay(mem2d, exp, wp, env, pos)
    samples = samples.reshape(_SAMPLE_B, _MBL, _FEAT)
    return samples, importance_weights


# speedup vs baseline: 1.0609x; 1.0609x over previous
import functools

import jax
import jax.numpy as jnp
from jax import lax
from jax.experimental import pallas as pl
from jax.experimental.pallas import tpu as pltpu
from jax.experimental.pallas import tpu_sc as plsc

_NUM_ENVS = 512
_MAX_LENGTH = 2048
_FEAT = 128
_SAMPLE_B = 1024
_MBL = 8
_L = 16
_NW = 32
_BPW = _SAMPLE_B // _NW
_RPW = _BPW * _MBL
_NCHUNK = 2
_CROWS = _RPW // _NCHUNK
_CSMP = _BPW // _NCHUNK


def _replay_body(mem_hbm, exp_hbm, wp_hbm, env_hbm, pos_hbm,
                 out_hbm, wout_hbm,
                 env_v, pos_v, wp_v, base_v, idx_v, rows_v, exp_v, o_v,
                 ones_v, sems):
    cid = lax.axis_index("c")
    sid = lax.axis_index("s")
    wid = sid * 2 + cid
    base = wid * _BPW

    cp_env = pltpu.async_copy(env_hbm.at[pl.ds(base, _BPW)], env_v, sems.at[0])
    cp_pos = pltpu.async_copy(pos_hbm.at[pl.ds(base, _BPW)], pos_v, sems.at[1])
    cp_wp = pltpu.async_copy(wp_hbm, wp_v, sems.at[2])

    lane = lax.iota(jnp.int32, _L)
    one16 = jnp.full((_L,), 1.0, jnp.float32)
    ones_v[pl.ds(0, _L)] = one16
    ones_v[pl.ds(_L, _L)] = one16
    cp_w = pltpu.async_copy(ones_v, wout_hbm.at[pl.ds(base, _BPW)], sems.at[4])

    cp_env.wait()
    cp_exp = pltpu.async_copy(exp_hbm.at[env_v], exp_v,
                              sems.at[5 + 2 * _NCHUNK])
    cp_pos.wait()

    for h in range(_BPW // _L):
        e16 = env_v[pl.ds(h * _L, _L)]
        p16 = pos_v[pl.ds(h * _L, _L)]
        base_v[pl.ds(h * _L, _L)] = e16 * _MAX_LENGTH + p16

    def idx_step(i, carry):
        j = lane + i * _L
        b = j >> 3
        t = j & 7
        idx_v[i // (_CROWS // _L), pl.ds((i % (_CROWS // _L)) * _L, _L)] = (
            plsc.load_gather(base_v, [b]) + t)
        return carry

    lax.fori_loop(0, _RPW // _L, idx_step, 0)
    gcps = []
    for k in range(_NCHUNK):
        gcps.append(pltpu.async_copy(
            mem_hbm.at[idx_v.at[k]],
            rows_v.at[pl.ds(k * _CROWS, _CROWS)], sems.at[5 + k]))

    cp_wp.wait()
    nhit = []
    for h in range(_BPW // _L):
        e16 = env_v[pl.ds(h * _L, _L)]
        p16 = pos_v[pl.ds(h * _L, _L)]
        o16 = plsc.load_gather(wp_v, [e16]) - p16
        o_v[pl.ds(_L + h * _L, _L)] = o16
        hit16 = (o16 >= 0) & (o16 < _MBL)
        nhit.append(jnp.max(jnp.where(hit16, 1, 0)))
    cp_exp.wait()

    def fixup(k):
        def body():
            def one_sample(smp, carry):
                sel = jnp.full((_L,), smp + _L, jnp.int32)
                o_spl = plsc.load_gather(o_v, [sel])
                hit = (o_spl >= 0) & (o_spl < _MBL)
                row_idx = smp * _MBL + jnp.where(hit, o_spl, 0)
                row_sel = jnp.full((_L,), smp, jnp.int32)
                for cc in range(_FEAT // _L):
                    col = lane + cc * _L
                    vals = plsc.load_gather(exp_v, [row_sel, col])
                    plsc.store_scatter(rows_v, [row_idx, col], vals, mask=hit)
                return carry
            lax.fori_loop(k * _CSMP, (k + 1) * _CSMP, one_sample, 0)
        pl.when(nhit[(k * _CSMP) // _L] > 0)(body)

    out_cps = []
    for k in range(_NCHUNK):
        gcps[k].wait()
        fixup(k)
        out_cps.append(pltpu.async_copy(
            rows_v.at[pl.ds(k * _CROWS, _CROWS)],
            out_hbm.at[pl.ds(wid * _RPW + k * _CROWS, _CROWS)],
            sems.at[5 + _NCHUNK + k]))
    for cp in out_cps:
        cp.wait()
    cp_w.wait()


@jax.jit
def _replay(mem2d, exp, write_pos, env_ids, positions):
    mesh = plsc.VectorSubcoreMesh(core_axis_name="c", subcore_axis_name="s",
                                  num_cores=2, num_subcores=16)
    run = pl.kernel(
        _replay_body,
        out_type=(
            jax.ShapeDtypeStruct((_SAMPLE_B * _MBL, _FEAT), jnp.float32),
            jax.ShapeDtypeStruct((_SAMPLE_B,), jnp.float32),
        ),
        mesh=mesh,
        compiler_params=pltpu.CompilerParams(needs_layout_passes=False,
                                             skip_device_barrier=True),
        scratch_types=[
            pltpu.VMEM((_BPW,), jnp.int32),
            pltpu.VMEM((_BPW,), jnp.int32),
            pltpu.VMEM((_NUM_ENVS,), jnp.int32),
            pltpu.VMEM((_BPW,), jnp.int32),
            pltpu.VMEM((_NCHUNK, _CROWS), jnp.int32),
            pltpu.VMEM((_RPW, _FEAT), jnp.float32),
            pltpu.VMEM((_BPW, _FEAT), jnp.float32),
            pltpu.VMEM((_L + _BPW,), jnp.int32),
            pltpu.VMEM((_BPW,), jnp.float32),
            pltpu.SemaphoreType.DMA((6 + 2 * _NCHUNK,)),
        ],
    )
    return run(mem2d, exp, write_pos, env_ids, positions)


def kernel(mem, exp, write_pos, env_ids, positions, mini_batch_length):
    mem2d = mem.reshape(_NUM_ENVS * _MAX_LENGTH, _FEAT)
    env = env_ids.astype(jnp.int32)
    wp = write_pos.astype(jnp.int32)
    pos = (positions + (mini_batch_length - _MBL)).astype(jnp.int32)
    samples, importance_weights = _replay(mem2d, exp, wp, env, pos)
    samples = samples.reshape(_SAMPLE_B, _MBL, _FEAT)
    return samples, importance_weights


# speedup vs baseline: 1.0618x; 1.0008x over previous
import functools

import jax
import jax.numpy as jnp
from jax import lax
from jax.experimental import pallas as pl
from jax.experimental.pallas import tpu as pltpu
from jax.experimental.pallas import tpu_sc as plsc

_NUM_ENVS = 512
_MAX_LENGTH = 2048
_FEAT = 128
_SAMPLE_B = 1024
_MBL = 8
_L = 16
_NW = 32
_BPW = _SAMPLE_B // _NW
_RPW = _BPW * _MBL
_NCHUNK = 2
_CROWS = _RPW // _NCHUNK
_CSMP = _BPW // _NCHUNK


def _replay_body(mem_hbm, exp_hbm, wp_hbm, env_hbm, pos_hbm,
                 out_hbm, wout_hbm,
                 env_v, pos_v, wp_v, base_v, idx_v, rows_v, exp_v, o_v,
                 ones_v, sems):
    cid = lax.axis_index("c")
    sid = lax.axis_index("s")
    wid = sid * 2 + cid
    base = wid * _BPW

    cp_env = pltpu.async_copy(env_hbm.at[pl.ds(base, _BPW)], env_v, sems.at[0])
    cp_pos = pltpu.async_copy(pos_hbm.at[pl.ds(base, _BPW)], pos_v, sems.at[1])
    cp_wp = pltpu.async_copy(wp_hbm, wp_v, sems.at[2])

    lane = lax.iota(jnp.int32, _L)
    one16 = jnp.full((_L,), 1.0, jnp.float32)
    ones_v[pl.ds(0, _L)] = one16
    ones_v[pl.ds(_L, _L)] = one16
    cp_w = pltpu.async_copy(ones_v, wout_hbm.at[pl.ds(base, _BPW)], sems.at[4])

    cp_env.wait()
    cp_exp = pltpu.async_copy(exp_hbm.at[env_v], exp_v,
                              sems.at[5 + 2 * _NCHUNK])
    cp_pos.wait()

    for h in range(_BPW // _L):
        e16 = env_v[pl.ds(h * _L, _L)]
        p16 = pos_v[pl.ds(h * _L, _L)]
        base_v[pl.ds(h * _L, _L)] = e16 * _MAX_LENGTH + p16

    def idx_step(i, carry):
        j = lane + i * _L
        b = j >> 3
        t = j & 7
        idx_v[i // (_CROWS // _L), pl.ds((i % (_CROWS // _L)) * _L, _L)] = (
            plsc.load_gather(base_v, [b]) + t)
        return carry

    lax.fori_loop(0, _RPW // _L, idx_step, 0)
    gcps = []
    for k in range(_NCHUNK):
        gcps.append(pltpu.async_copy(
            mem_hbm.at[idx_v.at[k]],
            rows_v.at[pl.ds(k * _CROWS, _CROWS)], sems.at[5 + k]))

    cp_wp.wait()
    nhit = []
    for h in range(_BPW // _L):
        e16 = env_v[pl.ds(h * _L, _L)]
        p16 = pos_v[pl.ds(h * _L, _L)]
        o16 = plsc.load_gather(wp_v, [e16]) - p16
        o_v[pl.ds(_L + h * _L, _L)] = o16
        hit16 = (o16 >= 0) & (o16 < _MBL)
        nhit.append(jnp.max(jnp.where(hit16, 1, 0)))
    cp_exp.wait()

    def fixup(k):
        def body():
            def one_sample(smp, carry):
                sel = jnp.full((_L,), smp + _L, jnp.int32)
                o_spl = plsc.load_gather(o_v, [sel])
                hit = (o_spl >= 0) & (o_spl < _MBL)
                row_idx = smp * _MBL + jnp.where(hit, o_spl, 0)
                row_sel = jnp.full((_L,), smp, jnp.int32)
                for cc in range(_FEAT // _L):
                    col = lane + cc * _L
                    vals = plsc.load_gather(exp_v, [row_sel, col])
                    plsc.store_scatter(rows_v, [row_idx, col], vals, mask=hit)
                return carry
            lax.fori_loop(k * _CSMP, (k + 1) * _CSMP, one_sample, 0)
        pl.when(nhit[(k * _CSMP) // _L] > 0)(body)

    for k in range(_NCHUNK):
        gcps[k].wait()
        fixup(k)
    pltpu.sync_copy(rows_v, out_hbm.at[pl.ds(wid * _RPW, _RPW)])
    cp_w.wait()


@jax.jit
def _replay(mem2d, exp, write_pos, env_ids, positions):
    mesh = plsc.VectorSubcoreMesh(core_axis_name="c", subcore_axis_name="s",
                                  num_cores=2, num_subcores=16)
    run = pl.kernel(
        _replay_body,
        out_type=(
            jax.ShapeDtypeStruct((_SAMPLE_B * _MBL, _FEAT), jnp.float32),
            jax.ShapeDtypeStruct((_SAMPLE_B,), jnp.float32),
        ),
        mesh=mesh,
        compiler_params=pltpu.CompilerParams(needs_layout_passes=False,
                                             skip_device_barrier=True),
        scratch_types=[
            pltpu.VMEM((_BPW,), jnp.int32),
            pltpu.VMEM((_BPW,), jnp.int32),
            pltpu.VMEM((_NUM_ENVS,), jnp.int32),
            pltpu.VMEM((_BPW,), jnp.int32),
            pltpu.VMEM((_NCHUNK, _CROWS), jnp.int32),
            pltpu.VMEM((_RPW, _FEAT), jnp.float32),
            pltpu.VMEM((_BPW, _FEAT), jnp.float32),
            pltpu.VMEM((_L + _BPW,), jnp.int32),
            pltpu.VMEM((_BPW,), jnp.float32),
            pltpu.SemaphoreType.DMA((6 + 2 * _NCHUNK,)),
        ],
    )
    return run(mem2d, exp, write_pos, env_ids, positions)


def kernel(mem, exp, write_pos, env_ids, positions, mini_batch_length):
    mem2d = mem.reshape(_NUM_ENVS * _MAX_LENGTH, _FEAT)
    env = env_ids.astype(jnp.int32)
    wp = write_pos.astype(jnp.int32)
    pos = (positions + (mini_batch_length - _MBL)).astype(jnp.int32)
    samples, importance_weights = _replay(mem2d, exp, wp, env, pos)
    samples = samples.reshape(_SAMPLE_B, _MBL, _FEAT)
    return samples, importance_weights


# speedup vs baseline: 1.0662x; 1.0042x over previous
import jax
import jax.numpy as jnp
from jax import lax
from jax.experimental import pallas as pl
from jax.experimental.pallas import tpu as pltpu
from jax.experimental.pallas import tpu_sc as plsc

_NUM_ENVS = 512
_MAX_LENGTH = 2048
_FEAT = 128
_SAMPLE_B = 1024
_MBL = 8
_L = 16
_NW = 32
_BPW = _SAMPLE_B // _NW
_RPW = _BPW * _MBL
_NCHUNK = 2
_CROWS = _RPW // _NCHUNK
_CSMP = _BPW // _NCHUNK


def _replay_body(mem_hbm, exp_hbm, wp_hbm, env_hbm, pos_hbm,
                 out_hbm, wout_hbm,
                 env_v, pos_v, wp_v, base_v, idx_v, rows_v, exp_v, o_v,
                 ones_v, sems):
    cid = lax.axis_index("c")
    sid = lax.axis_index("s")
    wid = sid * 2 + cid
    base = wid * _BPW

    cp_env = pltpu.async_copy(env_hbm.at[pl.ds(base, _BPW)], env_v, sems.at[0])
    cp_pos = pltpu.async_copy(pos_hbm.at[pl.ds(base, _BPW)], pos_v, sems.at[1])
    cp_wp = pltpu.async_copy(wp_hbm, wp_v, sems.at[2])

    lane = lax.iota(jnp.int32, _L)
    one16 = jnp.full((_L,), 1.0, jnp.float32)
    ones_v[pl.ds(0, _L)] = one16
    ones_v[pl.ds(_L, _L)] = one16
    cp_w = pltpu.async_copy(ones_v, wout_hbm.at[pl.ds(base, _BPW)], sems.at[4])

    cp_env.wait()
    cp_exp = pltpu.async_copy(exp_hbm.at[env_v], exp_v, sems.at[3])
    cp_pos.wait()

    for h in range(_BPW // _L):
        e16 = env_v[pl.ds(h * _L, _L)]
        p16 = pos_v[pl.ds(h * _L, _L)]
        base_v[pl.ds(h * _L, _L)] = e16 * _MAX_LENGTH + p16

    def idx_step(i, carry):
        j = lane + i * _L
        b = j >> 3
        t = j & 7
        idx_v[i // (_CROWS // _L), pl.ds((i % (_CROWS // _L)) * _L, _L)] = (
            plsc.load_gather(base_v, [b]) + t)
        return carry

    lax.fori_loop(0, _RPW // _L, idx_step, 0)
    gcps = []
    for k in range(_NCHUNK):
        gcps.append(pltpu.async_copy(
            mem_hbm.at[idx_v.at[k]],
            rows_v.at[pl.ds(k * _CROWS, _CROWS)], sems.at[5 + k]))

    cp_wp.wait()
    nhit = []
    for h in range(_BPW // _L):
        e16 = env_v[pl.ds(h * _L, _L)]
        p16 = pos_v[pl.ds(h * _L, _L)]
        o16 = plsc.load_gather(wp_v, [e16]) - p16
        o_v[pl.ds(_L + h * _L, _L)] = o16
        hit16 = (o16 >= 0) & (o16 < _MBL)
        nhit.append(jnp.max(jnp.where(hit16, 1, 0)))
    cp_exp.wait()

    def fixup(k):
        def body():
            def one_sample(smp, carry):
                sel = jnp.full((_L,), smp + _L, jnp.int32)
                o_spl = plsc.load_gather(o_v, [sel])
                hit = (o_spl >= 0) & (o_spl < _MBL)
                row_idx = smp * _MBL + jnp.where(hit, o_spl, 0)
                row_sel = jnp.full((_L,), smp, jnp.int32)
                for cc in range(_FEAT // _L):
                    col = lane + cc * _L
                    vals = plsc.load_gather(exp_v, [row_sel, col])
                    plsc.store_scatter(rows_v, [row_idx, col], vals, mask=hit)
                return carry
            lax.fori_loop(k * _CSMP, (k + 1) * _CSMP, one_sample, 0)
        pl.when(nhit[(k * _CSMP) // _L] > 0)(body)

    for k in range(_NCHUNK):
        gcps[k].wait()
        fixup(k)
    pltpu.sync_copy(rows_v, out_hbm.at[pl.ds(wid * _RPW, _RPW)])
    cp_w.wait()


@jax.jit
def _replay(mem2d, exp, write_pos, env_ids, positions):
    mesh = plsc.VectorSubcoreMesh(core_axis_name="c", subcore_axis_name="s",
                                  num_cores=2, num_subcores=16)
    run = pl.kernel(
        _replay_body,
        out_type=(
            jax.ShapeDtypeStruct((_SAMPLE_B * _MBL, _FEAT), jnp.float32),
            jax.ShapeDtypeStruct((_SAMPLE_B,), jnp.float32),
        ),
        mesh=mesh,
        compiler_params=pltpu.CompilerParams(needs_layout_passes=False,
                                             skip_device_barrier=True),
        scratch_types=[
            pltpu.VMEM((_BPW,), jnp.int32),
            pltpu.VMEM((_BPW,), jnp.int32),
            pltpu.VMEM((_NUM_ENVS,), jnp.int32),
            pltpu.VMEM((_BPW,), jnp.int32),
            pltpu.VMEM((_NCHUNK, _CROWS), jnp.int32),
            pltpu.VMEM((_RPW, _FEAT), jnp.float32),
            pltpu.VMEM((_BPW, _FEAT), jnp.float32),
            pltpu.VMEM((_L + _BPW,), jnp.int32),
            pltpu.VMEM((_BPW,), jnp.float32),
            pltpu.SemaphoreType.DMA((5 + _NCHUNK,)),
        ],
    )
    return run(mem2d, exp, write_pos, env_ids, positions)


def kernel(mem, exp, write_pos, env_ids, positions, mini_batch_length):
    mem2d = mem.reshape(_NUM_ENVS * _MAX_LENGTH, _FEAT)
    env = env_ids.astype(jnp.int32)
    wp = write_pos.astype(jnp.int32)
    pos = (positions + (mini_batch_length - _MBL)).astype(jnp.int32)
    samples, importance_weights = _replay(mem2d, exp, wp, env, pos)
    samples = samples.reshape(_SAMPLE_B, _MBL, _FEAT)
    return samples, importance_weights
